# baseline (speedup 1.0000x reference)
import functools

import jax
import jax.numpy as jnp
from jax import lax
from jax.experimental import pallas as pl
from jax.experimental.pallas import tpu as pltpu
from jax.experimental.pallas import tpu_sc as plsc

F32 = jnp.float32
I32 = jnp.int32

D_MODEL = 768
NHEAD = 12
DH = 64
D_FFN = 3072
NHASH = 4
BUCKET = 64
B = 2
SEQ = 4096
BH = B * NHEAD
NBUCKETS = SEQ // BUCKET
NTASK = BH * NHASH
NT = NTASK * SEQ
NCH = SEQ // 128
CPB = NHASH * NBUCKETS
NW = 32
TPW = NTASK // NW


def _qkv_body(x_ref, w_ref, o_ref):
    o_ref[...] = jnp.dot(x_ref[...], w_ref[...], preferred_element_type=F32)


def _qkv_call(x2, wcat):
    return pl.pallas_call(
        _qkv_body,
        grid=(B * SEQ // 1024,),
        in_specs=[
            pl.BlockSpec((1024, D_MODEL), lambda i: (i, 0)),
            pl.BlockSpec((D_MODEL, 2 * D_MODEL), lambda i: (0, 0)),
        ],
        out_specs=pl.BlockSpec((1024, 2 * D_MODEL), lambda i: (i, 0)),
        out_shape=jax.ShapeDtypeStruct((B * SEQ, 2 * D_MODEL), F32),
    )(x2, wcat)


def _bpos_body(qkT_ref, rotT_ref, p_ref):
    bh = pl.program_id(0)
    qT = qkT_ref[0]
    rT = jnp.dot(rotT_ref[...], qT, preferred_element_type=F32)

    sub_iota = lax.broadcasted_iota(I32, (32, SEQ), 0)
    buck_iota = lax.broadcasted_iota(I32, (NBUCKETS, SEQ), 0)
    triu = (lax.broadcasted_iota(I32, (128, 128), 0)
            < lax.broadcasted_iota(I32, (128, 128), 1)).astype(jnp.bfloat16)
    trilb = (lax.broadcasted_iota(I32, (NBUCKETS, NBUCKETS), 0)
             > lax.broadcasted_iota(I32, (NBUCKETS, NBUCKETS), 1)).astype(F32)

    for r in range(NHASH):
        rr = rT[32 * r:32 * (r + 1), :]
        m1 = jnp.max(rr, axis=0, keepdims=True)
        a1 = jnp.min(jnp.where(rr >= m1, sub_iota, 64), axis=0, keepdims=True)
        m2n = jnp.min(rr, axis=0, keepdims=True)
        a2 = jnp.min(jnp.where(rr <= m2n, sub_iota, 64), axis=0, keepdims=True)
        bucket = jnp.where(m1 >= -m2n, a1, 32 + a2)
        oh = (bucket == buck_iota).astype(F32)
        ohb = oh.astype(jnp.bfloat16)

        prefix = jnp.zeros((NBUCKETS, 1), F32)
        ranks = []
        for blk in range(SEQ // 128):
            ohf = oh[:, 128 * blk:128 * (blk + 1)]
            s = jnp.dot(ohb[:, 128 * blk:128 * (blk + 1)], triu,
                        preferred_element_type=F32)
            ranks.append(jnp.sum(ohf * (s + prefix), axis=0, keepdims=True))
            prefix = prefix + jnp.sum(ohf, axis=1, keepdims=True)
        rank = jnp.concatenate(ranks, axis=1)
        starts = jnp.dot(trilb, prefix, preferred_element_type=F32)
        startv = jnp.sum(oh * starts, axis=0, keepdims=True)
        base = (bh * NHASH + r) * SEQ
        pos = (rank + startv).astype(I32) + base
        p_ref[0, r] = pos[0]


def _bpos_call(qkT, rotT):
    return pl.pallas_call(
        _bpos_body,
        grid=(BH,),
        in_specs=[
            pl.BlockSpec((1, DH, SEQ), lambda i: (i, 0, 0)),
            pl.BlockSpec((NHASH * 32, DH), lambda i: (0, 0)),
        ],
        out_specs=pl.BlockSpec((1, NHASH, SEQ), lambda i: (i, 0, 0)),
        out_shape=jax.ShapeDtypeStruct((BH, NHASH, SEQ), I32),
    )(qkT, rotT)


def _make_sc_permute():
    mesh = plsc.VectorSubcoreMesh(core_axis_name="c", subcore_axis_name="s")

    @functools.partial(
        pl.kernel, mesh=mesh,
        compiler_params=pltpu.CompilerParams(needs_layout_passes=False),
        out_type=(jax.ShapeDtypeStruct((NT, 2 * DH), F32),
                  jax.ShapeDtypeStruct((NTASK, SEQ), I32)),
        scratch_types=(pltpu.VMEM((NCH, 128), I32),
                       pltpu.VMEM((SEQ,), I32),
                       pltpu.VMEM((4, 128, 2 * DH), F32),
                       pltpu.SemaphoreType.DMA,
                       pltpu.SemaphoreType.DMA),
    )
    def sc_permute(qv_hbm, p_hbm, sqkv_hbm, st_hbm, idx_v, st_v, buf_v,
                   sem_i, sem_o):
        wid = lax.axis_index("s") * 2 + lax.axis_index("c")
        for t in range(TPW):
            tid = wid * TPW + t
            bh = tid // NHASH
            base = tid * SEQ
            pltpu.sync_copy(p_hbm.at[tid], idx_v)

            ins = [None] * NCH
            outs = [None] * NCH
            for cc in range(NCH):
                if cc >= 4:
                    outs[cc - 4].wait()
                ins[cc] = pltpu.async_copy(qv_hbm.at[bh * NCH + cc],
                                           buf_v.at[cc % 4], sem_i)
                if cc >= 1:
                    ins[cc - 1].wait()
                    outs[cc - 1] = pltpu.async_copy(
                        buf_v.at[(cc - 1) % 4],
                        sqkv_hbm.at[idx_v.at[cc - 1]], sem_o)

            def _inv(g, carry):
                loc = idx_v[g // 8, pl.ds((g % 8) * 16, 16)] - base
                plsc.store_scatter(st_v, [loc],
                                   g * 16 + lax.iota(I32, 16))
                return carry
            lax.fori_loop(0, SEQ // 16, _inv, 0)
            pltpu.sync_copy(st_v, st_hbm.at[tid])

            ins[NCH - 1].wait()
            outs[NCH - 1] = pltpu.async_copy(
                buf_v.at[(NCH - 1) % 4],
                sqkv_hbm.at[idx_v.at[NCH - 1]], sem_o)
            for cc in range(NCH - 4, NCH):
                outs[cc].wait()

    return sc_permute


def _make_sc_unpermute():
    mesh = plsc.VectorSubcoreMesh(core_axis_name="c", subcore_axis_name="s")

    @functools.partial(
        pl.kernel, mesh=mesh,
        compiler_params=pltpu.CompilerParams(needs_layout_passes=False),
        out_type=jax.ShapeDtypeStruct((NTASK * NCH, 128, 2 * DH), F32),
        scratch_types=(pltpu.VMEM((NCH, 128), I32),
                       pltpu.VMEM((4, 128, 2 * DH), F32),
                       pltpu.SemaphoreType.DMA,
                       pltpu.SemaphoreType.DMA),
    )
    def sc_unpermute(so_hbm, p_hbm, otok_hbm, idx_v, buf_v, sem_i, sem_o):
        wid = lax.axis_index("s") * 2 + lax.axis_index("c")
        for t in range(TPW):
            tid = wid * TPW + t
            pltpu.sync_copy(p_hbm.at[tid], idx_v)
            ins = [None] * NCH
            outs = [None] * NCH
            for cc in range(NCH):
                if cc >= 4:
                    outs[cc - 4].wait()
                ins[cc] = pltpu.async_copy(so_hbm.at[idx_v.at[cc]],
                                           buf_v.at[cc % 4], sem_i)
                if cc >= 1:
                    ins[cc - 1].wait()
                    outs[cc - 1] = pltpu.async_copy(
                        buf_v.at[(cc - 1) % 4],
                        otok_hbm.at[tid * NCH + cc - 1], sem_o)
            ins[NCH - 1].wait()
            outs[NCH - 1] = pltpu.async_copy(
                buf_v.at[(NCH - 1) % 4],
                otok_hbm.at[tid * NCH + NCH - 1], sem_o)
            for cc in range(NCH - 4, NCH):
                outs[cc].wait()

    return sc_unpermute


def _sc_permute(qv4, p96):
    return _make_sc_permute()(qv4, p96)


def _sc_unpermute(so2, p96):
    return _make_sc_unpermute()(so2, p96)


_CG = 32


def _attn_body(sqv_ref, st_ref, pqv_ref, pst_ref, so_ref):
    scale = DH ** -0.5
    nrow = 64 * _CG
    a = sqv_ref[...]
    p = pqv_ref[...]
    q = a[:, :DH]
    n = jnp.sqrt(jnp.sum(q * q, axis=1, keepdims=True))
    kn = q / jnp.maximum(n, 1e-6)
    pk = p[:, :DH]
    pn = jnp.sqrt(jnp.sum(pk * pk, axis=1, keepdims=True))
    pkn = pk / jnp.maximum(pn, 1e-6)
    kn_s = jnp.concatenate([pkn, kn[:nrow - 64]], axis=0)
    v = a[:, DH:]
    v_s = jnp.concatenate([p[:, DH:], v[:nrow - 64]], axis=0)

    q3 = q.reshape(_CG, 64, DH)
    k6 = jnp.concatenate([kn.reshape(_CG, 64, DH),
                          kn_s.reshape(_CG, 64, DH)], axis=1)
    v6 = jnp.concatenate([v.reshape(_CG, 64, DH),
                          v_s.reshape(_CG, 64, DH)], axis=1)
    dots = lax.dot_general(q3, k6, (((2,), (2,)), ((0,), (0,))),
                           preferred_element_type=F32) * scale

    ii = lax.broadcasted_iota(I32, (_CG, 64, 128), 1)
    jj = lax.broadcasted_iota(I32, (_CG, 64, 128), 2)
    mask = ii == jj
    qid = st_ref[0, 0, pl.ds(0, 64)]
    kid = jnp.concatenate([qid, pst_ref[0, 0, 0, :]], axis=0)
    m0 = jnp.transpose(qid[None, :]) == kid[None, :]
    mask = jnp.concatenate([m0[None, :, :], mask[1:]], axis=0)

    dots = jnp.where(mask, -5e4, dots)
    m = jnp.max(dots, axis=2, keepdims=True)
    e = jnp.exp(dots - m)
    s = jnp.sum(e, axis=2, keepdims=True)
    lse = m + jnp.log(s)
    dexp = e * (1.0 / s)
    ob = lax.dot_general(dexp, v6, (((2,), (1,)), ((0,), (0,))),
                         preferred_element_type=F32)
    so_ref[...] = jnp.concatenate(
        [ob.reshape(nrow, DH),
         jnp.broadcast_to(lse, (_CG, 64, DH)).reshape(nrow, DH)], axis=1)


def _attn_call(sqkv, st3, st4):
    ppb = CPB // _CG

    def cur_idx(i, j):
        return (i * ppb + j, 0)

    def prev_idx(i, j):
        return (i * CPB + (j * _CG + CPB - 1) % CPB, 0)

    return pl.pallas_call(
        _attn_body,
        grid=(BH, ppb),
        in_specs=[
            pl.BlockSpec((64 * _CG, 2 * DH), cur_idx),
            pl.BlockSpec((1, 1, 64 * _CG), lambda i, j: (i, 0, j)),
            pl.BlockSpec((64, 2 * DH), prev_idx),
            pl.BlockSpec((1, 1, 1, 64),
                         lambda i, j: (i, (j * _CG + CPB - 1) % CPB, 0, 0)),
        ],
        out_specs=pl.BlockSpec((64 * _CG, 2 * DH), cur_idx),
        out_shape=jax.ShapeDtypeStruct((NT, 2 * DH), F32),
    )(sqkv, st3, sqkv, st4)


def _combine_body(o_ref, att_ref):
    ls = [o_ref[0, r][:, DH:DH + 1] for r in range(NHASH)]
    m = ls[0]
    for r in range(1, NHASH):
        m = jnp.maximum(m, ls[r])
    es = [jnp.exp(l - m) for l in ls]
    s = es[0]
    for r in range(1, NHASH):
        s = s + es[r]
    acc = o_ref[0, 0][:, :DH] * (es[0] / s)
    for r in range(1, NHASH):
        acc = acc + o_ref[0, r][:, :DH] * (es[r] / s)
    att_ref[0] = acc


def _combine_call(o_tok):
    return pl.pallas_call(
        _combine_body,
        grid=(BH,),
        in_specs=[
            pl.BlockSpec((1, NHASH, SEQ, 2 * DH), lambda i: (i, 0, 0, 0)),
        ],
        out_specs=pl.BlockSpec((1, SEQ, DH), lambda i: (i, 0, 0)),
        out_shape=jax.ShapeDtypeStruct((BH, SEQ, DH), F32),
    )(o_tok)


def _ln(x, g, b):
    m = jnp.mean(x, axis=-1, keepdims=True)
    v = jnp.mean((x - m) ** 2, axis=-1, keepdims=True)
    return (x - m) / jnp.sqrt(v + 1e-6) * g + b


def _make_ffn_body(final):
    def body(x_ref, a_ref, wo_ref, bo_ref, w1_ref, b1_ref, w2_ref, b2_ref,
             g2_ref, be2_ref, gf_ref, bf_ref, o_ref):
        z = x_ref[...] + jnp.dot(a_ref[...], wo_ref[...],
                                 preferred_element_type=F32) + bo_ref[...]
        h = jax.nn.relu(jnp.dot(z, w1_ref[...],
                                preferred_element_type=F32) + b1_ref[...])
        f = jnp.dot(h, w2_ref[...], preferred_element_type=F32) + b2_ref[...]
        y = _ln(z + f, g2_ref[...], be2_ref[...])
        if final:
            y = _ln(y, gf_ref[...], bf_ref[...])
        o_ref[...] = y
    return body


def _ffn_call(x2, att2, woT, bo2, w1T, b12, w2T, b22, g22, be22, gf2, bf2,
              final):
    tb = 1024
    row = lambda i: (i, 0)
    cst = lambda i: (0, 0)
    return pl.pallas_call(
        _make_ffn_body(final),
        grid=(B * SEQ // tb,),
        in_specs=[
            pl.BlockSpec((tb, D_MODEL), row),
            pl.BlockSpec((tb, D_MODEL), row),
            pl.BlockSpec((D_MODEL, D_MODEL), cst),
            pl.BlockSpec((1, D_MODEL), cst),
            pl.BlockSpec((D_MODEL, D_FFN), cst),
            pl.BlockSpec((1, D_FFN), cst),
            pl.BlockSpec((D_FFN, D_MODEL), cst),
            pl.BlockSpec((1, D_MODEL), cst),
            pl.BlockSpec((1, D_MODEL), cst),
            pl.BlockSpec((1, D_MODEL), cst),
            pl.BlockSpec((1, D_MODEL), cst),
            pl.BlockSpec((1, D_MODEL), cst),
        ],
        out_specs=pl.BlockSpec((tb, D_MODEL), row),
        out_shape=jax.ShapeDtypeStruct((B * SEQ, D_MODEL), F32),
    )(x2, att2, woT, bo2, w1T, b12, w2T, b22, g22, be22, gf2, bf2)


def kernel(src, Wqk, Wv, Wo, bo, W1, b1, W2, b2, g2, be2, gf, bf, rot):
    x2 = src.reshape(B * SEQ, D_MODEL)
    for i in range(2):
        wcat = jnp.concatenate([Wqk[i].T, Wv[i].T], axis=1)
        qkv = _qkv_call(x2, wcat)
        qkv4 = qkv.reshape(B, SEQ, 2, NHEAD, DH)
        qk_t = qkv4[:, :, 0].transpose(0, 2, 1, 3).reshape(BH, SEQ, DH)
        v_t = qkv4[:, :, 1].transpose(0, 2, 1, 3).reshape(BH, SEQ, DH)
        qkT = qk_t.transpose(0, 2, 1)
        rotT = rot[i].reshape(DH, NHASH * 32).T
        p = _bpos_call(qkT, rotT)
        p96 = p.reshape(NTASK, NCH, 128)
        qv4 = jnp.concatenate([qk_t, v_t], axis=-1).reshape(BH * NCH, 128,
                                                            2 * DH)
        sqkv, st = _sc_permute(qv4, p96)
        so2 = _attn_call(sqkv, st.reshape(BH, 1, NHASH * SEQ),
                         st.reshape(BH, CPB, 1, 64))
        o_tok = _sc_unpermute(so2, p96)
        att_t = _combine_call(o_tok.reshape(BH, NHASH, SEQ, 2 * DH))
        att2 = (att_t.reshape(B, NHEAD, SEQ, DH)
                .transpose(0, 2, 1, 3).reshape(B * SEQ, D_MODEL))
        x2 = _ffn_call(x2, att2, Wo[i].T, bo[i][None, :], W1[i].T,
                       b1[i][None, :], W2[i].T, b2[i][None, :],
                       g2[i][None, :], be2[i][None, :], gf[None, :],
                       bf[None, :], final=(i == 1))
    return x2.reshape(B, SEQ, D_MODEL)


# speedup vs baseline: 1.0483x; 1.0483x over previous
import functools

import jax
import jax.numpy as jnp
from jax import lax
from jax.experimental import pallas as pl
from jax.experimental.pallas import tpu as pltpu
from jax.experimental.pallas import tpu_sc as plsc

F32 = jnp.float32
I32 = jnp.int32

D_MODEL = 768
NHEAD = 12
DH = 64
D_FFN = 3072
NHASH = 4
BUCKET = 64
B = 2
SEQ = 4096
BH = B * NHEAD
NBUCKETS = SEQ // BUCKET
NTASK = BH * NHASH
NT = NTASK * SEQ
NCH = SEQ // 128
CPB = NHASH * NBUCKETS
NW = 32
TPW = NTASK // NW


def _qkv_body(x_ref, w_ref, o_ref):
    o_ref[...] = jnp.dot(x_ref[...], w_ref[...], preferred_element_type=F32)


def _qkv_call(x2, wcat):
    return pl.pallas_call(
        _qkv_body,
        grid=(B * SEQ // 1024,),
        in_specs=[
            pl.BlockSpec((1024, D_MODEL), lambda i: (i, 0)),
            pl.BlockSpec((D_MODEL, 2 * D_MODEL), lambda i: (0, 0)),
        ],
        out_specs=pl.BlockSpec((1024, 2 * D_MODEL), lambda i: (i, 0)),
        out_shape=jax.ShapeDtypeStruct((B * SEQ, 2 * D_MODEL), F32),
    )(x2, wcat)


def _bpos_body(qkT_ref, rotT_ref, p_ref):
    bh = pl.program_id(0)
    qT = qkT_ref[0]
    rT = jnp.dot(rotT_ref[...], qT, preferred_element_type=F32)

    sub_iota = lax.broadcasted_iota(I32, (32, SEQ), 0)
    buck_iota = lax.broadcasted_iota(I32, (NBUCKETS, SEQ), 0)
    triu = (lax.broadcasted_iota(I32, (128, 128), 0)
            < lax.broadcasted_iota(I32, (128, 128), 1)).astype(jnp.bfloat16)
    trilb = (lax.broadcasted_iota(I32, (NBUCKETS, NBUCKETS), 0)
             > lax.broadcasted_iota(I32, (NBUCKETS, NBUCKETS), 1)).astype(F32)

    for r in range(NHASH):
        rr = rT[32 * r:32 * (r + 1), :]
        m1 = jnp.max(rr, axis=0, keepdims=True)
        a1 = jnp.min(jnp.where(rr >= m1, sub_iota, 64), axis=0, keepdims=True)
        m2n = jnp.min(rr, axis=0, keepdims=True)
        a2 = jnp.min(jnp.where(rr <= m2n, sub_iota, 64), axis=0, keepdims=True)
        bucket = jnp.where(m1 >= -m2n, a1, 32 + a2)
        oh = (bucket == buck_iota).astype(F32)
        ohb = oh.astype(jnp.bfloat16)

        prefix = jnp.zeros((NBUCKETS, 1), F32)
        ranks = []
        for blk in range(SEQ // 128):
            ohf = oh[:, 128 * blk:128 * (blk + 1)]
            s = jnp.dot(ohb[:, 128 * blk:128 * (blk + 1)], triu,
                        preferred_element_type=F32)
            ranks.append(jnp.sum(ohf * (s + prefix), axis=0, keepdims=True))
            prefix = prefix + jnp.sum(ohf, axis=1, keepdims=True)
        rank = jnp.concatenate(ranks, axis=1)
        starts = jnp.dot(trilb, prefix, preferred_element_type=F32)
        startv = jnp.sum(oh * starts, axis=0, keepdims=True)
        base = (bh * NHASH + r) * SEQ
        pos = (rank + startv).astype(I32) + base
        p_ref[0, r] = pos[0]


def _bpos_call(qkT, rotT):
    return pl.pallas_call(
        _bpos_body,
        grid=(BH,),
        in_specs=[
            pl.BlockSpec((1, DH, SEQ), lambda i: (i, 0, 0)),
            pl.BlockSpec((NHASH * 32, DH), lambda i: (0, 0)),
        ],
        out_specs=pl.BlockSpec((1, NHASH, SEQ), lambda i: (i, 0, 0)),
        out_shape=jax.ShapeDtypeStruct((BH, NHASH, SEQ), I32),
    )(qkT, rotT)


def _make_sc_permute():
    mesh = plsc.VectorSubcoreMesh(core_axis_name="c", subcore_axis_name="s")

    @functools.partial(
        pl.kernel, mesh=mesh,
        compiler_params=pltpu.CompilerParams(needs_layout_passes=False),
        out_type=(jax.ShapeDtypeStruct((NT, 2 * DH), F32),
                  jax.ShapeDtypeStruct((NTASK, SEQ), I32)),
        scratch_types=(pltpu.VMEM((NCH, 128), I32),
                       pltpu.VMEM((SEQ,), I32),
                       pltpu.VMEM((4, 128, 2 * DH), F32),
                       pltpu.SemaphoreType.DMA,
                       pltpu.SemaphoreType.DMA),
    )
    def sc_permute(qv_hbm, p_hbm, sqkv_hbm, st_hbm, idx_v, st_v, buf_v,
                   sem_i, sem_o):
        wid = lax.axis_index("s") * 2 + lax.axis_index("c")
        for t in range(TPW):
            tid = wid * TPW + t
            bh = tid // NHASH
            b = bh // NHEAD
            h = bh % NHEAD
            base = tid * SEQ
            pltpu.sync_copy(p_hbm.at[tid], idx_v)

            ins = [None] * NCH
            outs = [None] * NCH
            for cc in range(NCH):
                if cc >= 4:
                    outs[cc - 4].wait()
                ins[cc] = pltpu.async_copy(
                    qv_hbm.at[b, pl.ds(cc * 128, 128), h],
                    buf_v.at[cc % 4], sem_i)
                if cc >= 1:
                    ins[cc - 1].wait()
                    outs[cc - 1] = pltpu.async_copy(
                        buf_v.at[(cc - 1) % 4],
                        sqkv_hbm.at[idx_v.at[cc - 1]], sem_o)

            def _inv(g, carry):
                loc = idx_v[g // 8, pl.ds((g % 8) * 16, 16)] - base
                plsc.store_scatter(st_v, [loc],
                                   g * 16 + lax.iota(I32, 16))
                return carry
            lax.fori_loop(0, SEQ // 16, _inv, 0)
            pltpu.sync_copy(st_v, st_hbm.at[tid])

            ins[NCH - 1].wait()
            outs[NCH - 1] = pltpu.async_copy(
                buf_v.at[(NCH - 1) % 4],
                sqkv_hbm.at[idx_v.at[NCH - 1]], sem_o)
            for cc in range(NCH - 4, NCH):
                outs[cc].wait()

    return sc_permute


def _make_sc_unpermute():
    mesh = plsc.VectorSubcoreMesh(core_axis_name="c", subcore_axis_name="s")

    @functools.partial(
        pl.kernel, mesh=mesh,
        compiler_params=pltpu.CompilerParams(needs_layout_passes=False),
        out_type=jax.ShapeDtypeStruct((NTASK * NCH, 128, 2 * DH), F32),
        scratch_types=(pltpu.VMEM((NCH, 128), I32),
                       pltpu.VMEM((4, 128, 2 * DH), F32),
                       pltpu.SemaphoreType.DMA,
                       pltpu.SemaphoreType.DMA),
    )
    def sc_unpermute(so_hbm, p_hbm, otok_hbm, idx_v, buf_v, sem_i, sem_o):
        wid = lax.axis_index("s") * 2 + lax.axis_index("c")
        for t in range(TPW):
            tid = wid * TPW + t
            pltpu.sync_copy(p_hbm.at[tid], idx_v)
            ins = [None] * NCH
            outs = [None] * NCH
            for cc in range(NCH):
                if cc >= 4:
                    outs[cc - 4].wait()
                ins[cc] = pltpu.async_copy(so_hbm.at[idx_v.at[cc]],
                                           buf_v.at[cc % 4], sem_i)
                if cc >= 1:
                    ins[cc - 1].wait()
                    outs[cc - 1] = pltpu.async_copy(
                        buf_v.at[(cc - 1) % 4],
                        otok_hbm.at[tid * NCH + cc - 1], sem_o)
            ins[NCH - 1].wait()
            outs[NCH - 1] = pltpu.async_copy(
                buf_v.at[(NCH - 1) % 4],
                otok_hbm.at[tid * NCH + NCH - 1], sem_o)
            for cc in range(NCH - 4, NCH):
                outs[cc].wait()

    return sc_unpermute


def _sc_permute(qv4, p96):
    return _make_sc_permute()(qv4, p96)


def _sc_unpermute(so2, p96):
    return _make_sc_unpermute()(so2, p96)


_CG = 32


def _attn_body(sqv_ref, st_ref, pqv_ref, pst_ref, so_ref):
    scale = DH ** -0.5
    nrow = 64 * _CG
    a = sqv_ref[...]
    p = pqv_ref[...]
    q = a[:, :DH]
    s2 = jnp.sum(q * q, axis=1, keepdims=True)
    kn = q * lax.rsqrt(jnp.maximum(s2, 1e-12))
    pk = p[:, :DH]
    pkn = pk * lax.rsqrt(jnp.maximum(
        jnp.sum(pk * pk, axis=1, keepdims=True), 1e-12))
    v = a[:, DH:]

    q3 = q.reshape(_CG, 64, DH)
    k3 = kn.reshape(_CG, 64, DH)
    k3s = jnp.concatenate([pkn[None], k3[:_CG - 1]], axis=0)
    v3 = v.reshape(_CG, 64, DH)
    v3s = jnp.concatenate([p[:, DH:][None], v3[:_CG - 1]], axis=0)
    dn = (((2,), (2,)), ((0,), (0,)))
    dots = jnp.concatenate(
        [lax.dot_general(q3, k3, dn, preferred_element_type=F32),
         lax.dot_general(q3, k3s, dn, preferred_element_type=F32)],
        axis=2) * scale

    ii = lax.broadcasted_iota(I32, (_CG, 64, 128), 1)
    jj = lax.broadcasted_iota(I32, (_CG, 64, 128), 2)
    mask = ii == jj
    qid = st_ref[0, 0, pl.ds(0, 64)]
    kid = jnp.concatenate([qid, pst_ref[0, 0, 0, :]], axis=0)
    m0 = jnp.transpose(qid[None, :]) == kid[None, :]
    mask = jnp.concatenate([m0[None, :, :], mask[1:]], axis=0)

    dots = jnp.where(mask, -5e4, dots)
    m = jnp.max(dots, axis=2, keepdims=True)
    lse = m + jnp.log(jnp.sum(jnp.exp(dots - m), axis=2, keepdims=True))
    dexp = jnp.exp(dots - lse)
    dn2 = (((2,), (1,)), ((0,), (0,)))
    ob = (lax.dot_general(dexp[:, :, :64], v3, dn2,
                          preferred_element_type=F32)
          + lax.dot_general(dexp[:, :, 64:], v3s, dn2,
                            preferred_element_type=F32))
    so_ref[...] = jnp.concatenate(
        [ob.reshape(nrow, DH),
         jnp.broadcast_to(lse, (_CG, 64, DH)).reshape(nrow, DH)], axis=1)


def _attn_call(sqkv, st3, st4):
    ppb = CPB // _CG

    def cur_idx(i, j):
        return (i * ppb + j, 0)

    def prev_idx(i, j):
        return (i * CPB + (j * _CG + CPB - 1) % CPB, 0)

    return pl.pallas_call(
        _attn_body,
        grid=(BH, ppb),
        in_specs=[
            pl.BlockSpec((64 * _CG, 2 * DH), cur_idx),
            pl.BlockSpec((1, 1, 64 * _CG), lambda i, j: (i, 0, j)),
            pl.BlockSpec((64, 2 * DH), prev_idx),
            pl.BlockSpec((1, 1, 1, 64),
                         lambda i, j: (i, (j * _CG + CPB - 1) % CPB, 0, 0)),
        ],
        out_specs=pl.BlockSpec((64 * _CG, 2 * DH), cur_idx),
        out_shape=jax.ShapeDtypeStruct((NT, 2 * DH), F32),
    )(sqkv, st3, sqkv, st4)


def _combine_body(o_ref, att_ref):
    halves = []
    for hh in range(2):
        ls = [o_ref[hh, r][:, DH:DH + 1] for r in range(NHASH)]
        m = ls[0]
        for r in range(1, NHASH):
            m = jnp.maximum(m, ls[r])
        es = [jnp.exp(l - m) for l in ls]
        s = es[0]
        for r in range(1, NHASH):
            s = s + es[r]
        acc = o_ref[hh, 0][:, :DH] * (es[0] / s)
        for r in range(1, NHASH):
            acc = acc + o_ref[hh, r][:, :DH] * (es[r] / s)
        halves.append(acc)
    att_ref[0] = jnp.concatenate(halves, axis=1)


def _combine_call(o_tok):
    return pl.pallas_call(
        _combine_body,
        grid=(BH // 2,),
        in_specs=[
            pl.BlockSpec((2, NHASH, SEQ, 2 * DH), lambda g: (g, 0, 0, 0)),
        ],
        out_specs=pl.BlockSpec((1, SEQ, 2 * DH),
                               lambda g: (g // (NHEAD // 2), 0,
                                          g % (NHEAD // 2))),
        out_shape=jax.ShapeDtypeStruct((B, SEQ, D_MODEL), F32),
    )(o_tok)


def _ln(x, g, b):
    m = jnp.mean(x, axis=-1, keepdims=True)
    v = jnp.mean((x - m) ** 2, axis=-1, keepdims=True)
    return (x - m) / jnp.sqrt(v + 1e-6) * g + b


def _make_ffn_body(final):
    def body(x_ref, a_ref, wo_ref, bo_ref, w1_ref, b1_ref, w2_ref, b2_ref,
             g2_ref, be2_ref, gf_ref, bf_ref, o_ref):
        z = x_ref[...] + jnp.dot(a_ref[...], wo_ref[...],
                                 preferred_element_type=F32) + bo_ref[...]
        h = jax.nn.relu(jnp.dot(z, w1_ref[...],
                                preferred_element_type=F32) + b1_ref[...])
        f = jnp.dot(h, w2_ref[...], preferred_element_type=F32) + b2_ref[...]
        y = _ln(z + f, g2_ref[...], be2_ref[...])
        if final:
            y = _ln(y, gf_ref[...], bf_ref[...])
        o_ref[...] = y
    return body


def _ffn_call(x2, att2, woT, bo2, w1T, b12, w2T, b22, g22, be22, gf2, bf2,
              final):
    tb = 1024
    row = lambda i: (i, 0)
    cst = lambda i: (0, 0)
    return pl.pallas_call(
        _make_ffn_body(final),
        grid=(B * SEQ // tb,),
        in_specs=[
            pl.BlockSpec((tb, D_MODEL), row),
            pl.BlockSpec((tb, D_MODEL), row),
            pl.BlockSpec((D_MODEL, D_MODEL), cst),
            pl.BlockSpec((1, D_MODEL), cst),
            pl.BlockSpec((D_MODEL, D_FFN), cst),
            pl.BlockSpec((1, D_FFN), cst),
            pl.BlockSpec((D_FFN, D_MODEL), cst),
            pl.BlockSpec((1, D_MODEL), cst),
            pl.BlockSpec((1, D_MODEL), cst),
            pl.BlockSpec((1, D_MODEL), cst),
            pl.BlockSpec((1, D_MODEL), cst),
            pl.BlockSpec((1, D_MODEL), cst),
        ],
        out_specs=pl.BlockSpec((tb, D_MODEL), row),
        out_shape=jax.ShapeDtypeStruct((B * SEQ, D_MODEL), F32),
    )(x2, att2, woT, bo2, w1T, b12, w2T, b22, g22, be22, gf2, bf2)


def kernel(src, Wqk, Wv, Wo, bo, W1, b1, W2, b2, g2, be2, gf, bf, rot):
    x2 = src.reshape(B * SEQ, D_MODEL)
    for i in range(2):
        wcat = jnp.concatenate(
            [Wqk[i].T.reshape(D_MODEL, NHEAD, DH),
             Wv[i].T.reshape(D_MODEL, NHEAD, DH)],
            axis=-1).reshape(D_MODEL, 2 * D_MODEL)
        qkv = _qkv_call(x2, wcat)
        qv5 = qkv.reshape(B, SEQ, NHEAD, 2 * DH)
        qkT = qv5[..., :DH].transpose(0, 2, 3, 1).reshape(BH, DH, SEQ)
        rotT = rot[i].reshape(DH, NHASH * 32).T
        p = _bpos_call(qkT, rotT)
        p96 = p.reshape(NTASK, NCH, 128)
        sqkv, st = _sc_permute(qv5, p96)
        so2 = _attn_call(sqkv, st.reshape(BH, 1, NHASH * SEQ),
                         st.reshape(BH, CPB, 1, 64))
        o_tok = _sc_unpermute(so2, p96)
        att2 = _combine_call(
            o_tok.reshape(BH, NHASH, SEQ, 2 * DH)).reshape(B * SEQ, D_MODEL)
        x2 = _ffn_call(x2, att2, Wo[i].T, bo[i][None, :], W1[i].T,
                       b1[i][None, :], W2[i].T, b2[i][None, :],
                       g2[i][None, :], be2[i][None, :], gf[None, :],
                       bf[None, :], final=(i == 1))
    return x2.reshape(B, SEQ, D_MODEL)


# speedup vs baseline: 1.2064x; 1.1508x over previous
import functools

import jax
import jax.numpy as jnp
from jax import lax
from jax.experimental import pallas as pl
from jax.experimental.pallas import tpu as pltpu
from jax.experimental.pallas import tpu_sc as plsc

F32 = jnp.float32
I32 = jnp.int32

D_MODEL = 768
NHEAD = 12
DH = 64
D_FFN = 3072
NHASH = 4
BUCKET = 64
B = 2
SEQ = 4096
BH = B * NHEAD
NBUCKETS = SEQ // BUCKET
NTASK = BH * NHASH
NT = NTASK * SEQ
NCH = SEQ // 128
CPB = NHASH * NBUCKETS
NW = 32
TPW = NTASK // NW


def _qkv_body(x_ref, w_ref, o_ref):
    o_ref[...] = jnp.dot(x_ref[...], w_ref[...], preferred_element_type=F32)


def _qkv_call(x2, wcat):
    return pl.pallas_call(
        _qkv_body,
        grid=(B * SEQ // 1024,),
        in_specs=[
            pl.BlockSpec((1024, D_MODEL), lambda i: (i, 0)),
            pl.BlockSpec((D_MODEL, 2 * D_MODEL), lambda i: (0, 0)),
        ],
        out_specs=pl.BlockSpec((1024, 2 * D_MODEL), lambda i: (i, 0)),
        out_shape=jax.ShapeDtypeStruct((B * SEQ, 2 * D_MODEL), F32),
    )(x2, wcat)


def _bpos_body(qkT_ref, rotT_ref, p_ref):
    bh = pl.program_id(0)
    qT = qkT_ref[0]
    rT = jnp.dot(rotT_ref[...], qT, preferred_element_type=F32)

    sub_iota = lax.broadcasted_iota(I32, (32, SEQ), 0)
    buck_iota = lax.broadcasted_iota(I32, (NBUCKETS, SEQ), 0)
    triu = (lax.broadcasted_iota(I32, (128, 128), 0)
            < lax.broadcasted_iota(I32, (128, 128), 1)).astype(jnp.bfloat16)
    trilb = (lax.broadcasted_iota(I32, (NBUCKETS, NBUCKETS), 0)
             > lax.broadcasted_iota(I32, (NBUCKETS, NBUCKETS), 1)).astype(F32)

    for r in range(NHASH):
        rr = rT[32 * r:32 * (r + 1), :]
        m1 = jnp.max(rr, axis=0, keepdims=True)
        a1 = jnp.min(jnp.where(rr >= m1, sub_iota, 64), axis=0, keepdims=True)
        m2n = jnp.min(rr, axis=0, keepdims=True)
        a2 = jnp.min(jnp.where(rr <= m2n, sub_iota, 64), axis=0, keepdims=True)
        bucket = jnp.where(m1 >= -m2n, a1, 32 + a2)
        oh = (bucket == buck_iota).astype(F32)
        ohb = oh.astype(jnp.bfloat16)

        prefix = jnp.zeros((NBUCKETS, 1), F32)
        ranks = []
        for blk in range(SEQ // 128):
            ohf = oh[:, 128 * blk:128 * (blk + 1)]
            s = jnp.dot(ohb[:, 128 * blk:128 * (blk + 1)], triu,
                        preferred_element_type=F32)
            ranks.append(jnp.sum(ohf * (s + prefix), axis=0, keepdims=True))
            prefix = prefix + jnp.sum(ohf, axis=1, keepdims=True)
        rank = jnp.concatenate(ranks, axis=1)
        starts = jnp.dot(trilb, prefix, preferred_element_type=F32)
        startv = jnp.sum(oh * starts, axis=0, keepdims=True)
        base = (bh * NHASH + r) * SEQ
        pos = (rank + startv).astype(I32) + base
        p_ref[0, r] = pos[0]


def _bpos_call(qkT, rotT):
    return pl.pallas_call(
        _bpos_body,
        grid=(BH,),
        in_specs=[
            pl.BlockSpec((1, DH, SEQ), lambda i: (i, 0, 0)),
            pl.BlockSpec((NHASH * 32, DH), lambda i: (0, 0)),
        ],
        out_specs=pl.BlockSpec((1, NHASH, SEQ), lambda i: (i, 0, 0)),
        out_shape=jax.ShapeDtypeStruct((BH, NHASH, SEQ), I32),
    )(qkT, rotT)


def _make_sc_permute():
    mesh = plsc.VectorSubcoreMesh(core_axis_name="c", subcore_axis_name="s")

    @functools.partial(
        pl.kernel, mesh=mesh,
        compiler_params=pltpu.CompilerParams(needs_layout_passes=False),
        out_type=(jax.ShapeDtypeStruct((NT, 2 * DH), F32),
                  jax.ShapeDtypeStruct((NTASK, SEQ), I32)),
        scratch_types=(pltpu.VMEM((NCH, 128), I32),
                       pltpu.VMEM((SEQ,), I32),
                       pltpu.VMEM((4, 128, 2 * DH), F32),
                       pltpu.SemaphoreType.DMA,
                       pltpu.SemaphoreType.DMA),
    )
    def sc_permute(qv_hbm, p_hbm, sqkv_hbm, st_hbm, idx_v, st_v, buf_v,
                   sem_i, sem_o):
        wid = lax.axis_index("s") * 2 + lax.axis_index("c")
        for t in range(TPW):
            tid = wid * TPW + t
            bh = tid // NHASH
            b = bh // NHEAD
            h = bh % NHEAD
            base = tid * SEQ
            pltpu.sync_copy(p_hbm.at[tid], idx_v)

            ins = [None] * NCH
            outs = [None] * NCH
            for cc in range(NCH):
                if cc >= 4:
                    outs[cc - 4].wait()
                ins[cc] = pltpu.async_copy(
                    qv_hbm.at[b, pl.ds(cc * 128, 128), h],
                    buf_v.at[cc % 4], sem_i)
                if cc >= 1:
                    ins[cc - 1].wait()
                    outs[cc - 1] = pltpu.async_copy(
                        buf_v.at[(cc - 1) % 4],
                        sqkv_hbm.at[idx_v.at[cc - 1]], sem_o)

            def _inv(g, carry):
                loc = idx_v[g // 8, pl.ds((g % 8) * 16, 16)] - base
                plsc.store_scatter(st_v, [loc],
                                   g * 16 + lax.iota(I32, 16))
                return carry
            lax.fori_loop(0, SEQ // 16, _inv, 0)
            pltpu.sync_copy(st_v, st_hbm.at[tid])

            ins[NCH - 1].wait()
            outs[NCH - 1] = pltpu.async_copy(
                buf_v.at[(NCH - 1) % 4],
                sqkv_hbm.at[idx_v.at[NCH - 1]], sem_o)
            for cc in range(NCH - 4, NCH):
                outs[cc].wait()

    return sc_permute


def _make_sc_unpermute():
    mesh = plsc.VectorSubcoreMesh(core_axis_name="c", subcore_axis_name="s")

    @functools.partial(
        pl.kernel, mesh=mesh,
        compiler_params=pltpu.CompilerParams(needs_layout_passes=False),
        out_type=jax.ShapeDtypeStruct((NTASK * NCH, 128, 2 * DH), F32),
        scratch_types=(pltpu.VMEM((NCH, 128), I32),
                       pltpu.VMEM((4, 128, 2 * DH), F32),
                       pltpu.SemaphoreType.DMA,
                       pltpu.SemaphoreType.DMA),
    )
    def sc_unpermute(so_hbm, p_hbm, otok_hbm, idx_v, buf_v, sem_i, sem_o):
        wid = lax.axis_index("s") * 2 + lax.axis_index("c")
        for t in range(TPW):
            tid = wid * TPW + t
            pltpu.sync_copy(p_hbm.at[tid], idx_v)
            ins = [None] * NCH
            outs = [None] * NCH
            for cc in range(NCH):
                if cc >= 4:
                    outs[cc - 4].wait()
                ins[cc] = pltpu.async_copy(so_hbm.at[idx_v.at[cc]],
                                           buf_v.at[cc % 4], sem_i)
                if cc >= 1:
                    ins[cc - 1].wait()
                    outs[cc - 1] = pltpu.async_copy(
                        buf_v.at[(cc - 1) % 4],
                        otok_hbm.at[tid * NCH + cc - 1], sem_o)
            ins[NCH - 1].wait()
            outs[NCH - 1] = pltpu.async_copy(
                buf_v.at[(NCH - 1) % 4],
                otok_hbm.at[tid * NCH + NCH - 1], sem_o)
            for cc in range(NCH - 4, NCH):
                outs[cc].wait()

    return sc_unpermute


def _sc_permute(qv4, p96):
    return _make_sc_permute()(qv4, p96)


def _sc_unpermute(so2, p96):
    return _make_sc_unpermute()(so2, p96)


_CG = 32


def _attn_body(sqv_ref, st_ref, pqv_ref, pst_ref, so_ref):
    scale = DH ** -0.5
    nrow = 64 * _CG
    a = sqv_ref[...]
    p = pqv_ref[...]
    q = a[:, :DH]
    n = jnp.sqrt(jnp.sum(q * q, axis=1, keepdims=True))
    kn = q / jnp.maximum(n, 1e-6)
    pk = p[:, :DH]
    pn = jnp.sqrt(jnp.sum(pk * pk, axis=1, keepdims=True))
    pkn = pk / jnp.maximum(pn, 1e-6)
    kn_s = jnp.concatenate([pkn, kn[:nrow - 64]], axis=0)
    v = a[:, DH:]
    v_s = jnp.concatenate([p[:, DH:], v[:nrow - 64]], axis=0)

    q3 = q.reshape(_CG, 64, DH)
    k6 = jnp.concatenate([kn.reshape(_CG, 64, DH),
                          kn_s.reshape(_CG, 64, DH)], axis=1)
    v6 = jnp.concatenate([v.reshape(_CG, 64, DH),
                          v_s.reshape(_CG, 64, DH)], axis=1)
    dots = lax.dot_general(q3, k6, (((2,), (2,)), ((0,), (0,))),
                           preferred_element_type=F32) * scale

    ii = lax.broadcasted_iota(I32, (_CG, 64, 128), 1)
    jj = lax.broadcasted_iota(I32, (_CG, 64, 128), 2)
    mask = ii == jj
    qid = st_ref[0, 0, pl.ds(0, 64)]
    kid = jnp.concatenate([qid, pst_ref[0, 0, 0, :]], axis=0)
    m0 = jnp.transpose(qid[None, :]) == kid[None, :]
    mask = jnp.concatenate([m0[None, :, :], mask[1:]], axis=0)

    dots = jnp.where(mask, -5e4, dots)
    m = jnp.max(dots, axis=2, keepdims=True)
    lse = m + jnp.log(jnp.sum(jnp.exp(dots - m), axis=2, keepdims=True))
    dexp = jnp.exp(dots - lse)
    ob = lax.dot_general(dexp, v6, (((2,), (1,)), ((0,), (0,))),
                         preferred_element_type=F32)
    so_ref[...] = jnp.concatenate(
        [ob.reshape(nrow, DH),
         jnp.broadcast_to(lse, (_CG, 64, DH)).reshape(nrow, DH)], axis=1)


def _attn_call(sqkv, st3, st4):
    ppb = CPB // _CG

    def cur_idx(i, j):
        return (i * ppb + j, 0)

    def prev_idx(i, j):
        return (i * CPB + (j * _CG + CPB - 1) % CPB, 0)

    return pl.pallas_call(
        _attn_body,
        grid=(BH, ppb),
        in_specs=[
            pl.BlockSpec((64 * _CG, 2 * DH), cur_idx),
            pl.BlockSpec((1, 1, 64 * _CG), lambda i, j: (i, 0, j)),
            pl.BlockSpec((64, 2 * DH), prev_idx),
            pl.BlockSpec((1, 1, 1, 64),
                         lambda i, j: (i, (j * _CG + CPB - 1) % CPB, 0, 0)),
        ],
        out_specs=pl.BlockSpec((64 * _CG, 2 * DH), cur_idx),
        out_shape=jax.ShapeDtypeStruct((NT, 2 * DH), F32),
    )(sqkv, st3, sqkv, st4)


def _combine_body(o_ref, att_ref):
    halves = []
    for hh in range(2):
        ls = [o_ref[hh, r][:, DH:DH + 1] for r in range(NHASH)]
        m = ls[0]
        for r in range(1, NHASH):
            m = jnp.maximum(m, ls[r])
        es = [jnp.exp(l - m) for l in ls]
        s = es[0]
        for r in range(1, NHASH):
            s = s + es[r]
        acc = o_ref[hh, 0][:, :DH] * (es[0] / s)
        for r in range(1, NHASH):
            acc = acc + o_ref[hh, r][:, :DH] * (es[r] / s)
        halves.append(acc)
    att_ref[0] = jnp.concatenate(halves, axis=1)


def _combine_call(o_tok):
    return pl.pallas_call(
        _combine_body,
        grid=(BH // 2,),
        in_specs=[
            pl.BlockSpec((2, NHASH, SEQ, 2 * DH), lambda g: (g, 0, 0, 0)),
        ],
        out_specs=pl.BlockSpec((1, SEQ, 2 * DH),
                               lambda g: (g // (NHEAD // 2), 0,
                                          g % (NHEAD // 2))),
        out_shape=jax.ShapeDtypeStruct((B, SEQ, D_MODEL), F32),
    )(o_tok)


def _ln(x, g, b):
    m = jnp.mean(x, axis=-1, keepdims=True)
    v = jnp.mean((x - m) ** 2, axis=-1, keepdims=True)
    return (x - m) / jnp.sqrt(v + 1e-6) * g + b


def _make_ffn_body(final):
    def body(x_ref, a_ref, wo_ref, bo_ref, w1_ref, b1_ref, w2_ref, b2_ref,
             g2_ref, be2_ref, gf_ref, bf_ref, o_ref):
        z = x_ref[...] + jnp.dot(a_ref[...], wo_ref[...],
                                 preferred_element_type=F32) + bo_ref[...]
        h = jax.nn.relu(jnp.dot(z, w1_ref[...],
                                preferred_element_type=F32) + b1_ref[...])
        f = jnp.dot(h, w2_ref[...], preferred_element_type=F32) + b2_ref[...]
        y = _ln(z + f, g2_ref[...], be2_ref[...])
        if final:
            y = _ln(y, gf_ref[...], bf_ref[...])
        o_ref[...] = y
    return body


def _ffn_call(x2, att2, woT, bo2, w1T, b12, w2T, b22, g22, be22, gf2, bf2,
              final):
    tb = 1024
    row = lambda i: (i, 0)
    cst = lambda i: (0, 0)
    return pl.pallas_call(
        _make_ffn_body(final),
        grid=(B * SEQ // tb,),
        in_specs=[
            pl.BlockSpec((tb, D_MODEL), row),
            pl.BlockSpec((tb, D_MODEL), row),
            pl.BlockSpec((D_MODEL, D_MODEL), cst),
            pl.BlockSpec((1, D_MODEL), cst),
            pl.BlockSpec((D_MODEL, D_FFN), cst),
            pl.BlockSpec((1, D_FFN), cst),
            pl.BlockSpec((D_FFN, D_MODEL), cst),
            pl.BlockSpec((1, D_MODEL), cst),
            pl.BlockSpec((1, D_MODEL), cst),
            pl.BlockSpec((1, D_MODEL), cst),
            pl.BlockSpec((1, D_MODEL), cst),
            pl.BlockSpec((1, D_MODEL), cst),
        ],
        out_specs=pl.BlockSpec((tb, D_MODEL), row),
        out_shape=jax.ShapeDtypeStruct((B * SEQ, D_MODEL), F32),
    )(x2, att2, woT, bo2, w1T, b12, w2T, b22, g22, be22, gf2, bf2)


def kernel(src, Wqk, Wv, Wo, bo, W1, b1, W2, b2, g2, be2, gf, bf, rot):
    x2 = src.reshape(B * SEQ, D_MODEL)
    for i in range(2):
        wcat = jnp.concatenate(
            [Wqk[i].T.reshape(D_MODEL, NHEAD, DH),
             Wv[i].T.reshape(D_MODEL, NHEAD, DH)],
            axis=-1).reshape(D_MODEL, 2 * D_MODEL)
        qkv = _qkv_call(x2, wcat)
        qv5 = qkv.reshape(B, SEQ, NHEAD, 2 * DH)
        qkT = qv5[..., :DH].transpose(0, 2, 3, 1).reshape(BH, DH, SEQ)
        rotT = rot[i].reshape(DH, NHASH * 32).T
        p = _bpos_call(qkT, rotT)
        p96 = p.reshape(NTASK, NCH, 128)
        sqkv, st = _sc_permute(qv5, p96)
        so2 = _attn_call(sqkv, st.reshape(BH, 1, NHASH * SEQ),
                         st.reshape(BH, CPB, 1, 64))
        o_tok = _sc_unpermute(so2, p96)
        att2 = _combine_call(
            o_tok.reshape(BH, NHASH, SEQ, 2 * DH)).reshape(B * SEQ, D_MODEL)
        x2 = _ffn_call(x2, att2, Wo[i].T, bo[i][None, :], W1[i].T,
                       b1[i][None, :], W2[i].T, b2[i][None, :],
                       g2[i][None, :], be2[i][None, :], gf[None, :],
                       bf[None, :], final=(i == 1))
    return x2.reshape(B, SEQ, D_MODEL)
